# Initial kernel scaffold; baseline (speedup 1.0000x reference)
import jax
import jax.numpy as jnp
from jax.experimental import pallas as pl


def kernel(x, em, W, b):
    raise NotImplementedError("write your pallas kernel here")


# speedup vs baseline: 14.5354x; 14.5354x over previous
import functools

import jax
import jax.numpy as jnp
from jax import lax
from jax.experimental import pallas as pl
from jax.experimental.pallas import tpu as pltpu
from jax.experimental.pallas import tpu_sc as plsc

NC, NS, LANES = 2, 16, 16
NW = NC * NS
B, L, D = 16384, 20, 16
ROWS_PER_W = B // NW
SUB = 64
NSUB = ROWS_PER_W // SUB
FLAT_PER_SUB = SUB * L
CHUNK = 128
NCHUNK = FLAT_PER_SUB // CHUNK
IDX_ROWS_PER_W = ROWS_PER_W * L // CHUNK
GROUPS = SUB // LANES


def _tree_sum(vs):
    while len(vs) > 1:
        nxt = [vs[i] + vs[i + 1] for i in range(0, len(vs) - 1, 2)]
        if len(vs) % 2:
            nxt.append(vs[-1])
        vs = nxt
    return vs[0]


def _sc_body(xr_hbm, em_hbm, w_hbm, bias_hbm, out_hbm,
             idx_v, rows_v, w_v, bias_v, out_v, sem):
    c = lax.axis_index("c")
    s = lax.axis_index("s")
    wid = s * NC + c
    iota = lax.iota(jnp.int32, LANES)

    pltpu.sync_copy(xr_hbm.at[pl.ds(wid * IDX_ROWS_PER_W, IDX_ROWS_PER_W)],
                    idx_v)
    pltpu.sync_copy(w_hbm, w_v)
    pltpu.sync_copy(bias_hbm, bias_v)

    def sub_block(sb, carry):
        cps = []
        for j in range(NCHUNK):
            cps.append(pltpu.async_copy(
                em_hbm.at[idx_v.at[sb * NCHUNK + j]],
                rows_v.at[pl.ds(j * CHUNK, CHUNK)],
                sem))
        for cp in cps:
            cp.wait()

        def group(g, carry2):
            base = g * (LANES * L)
            bias_vec = bias_v[...]
            tot = bias_vec
            for i in range(LANES):
                r0 = base + i * L
                acc = rows_v[r0, :] * w_v[0, :]
                for l in range(1, L):
                    acc = acc + rows_v[r0 + l, :] * w_v[l, :]
                for sh in (8, 4, 2, 1):
                    acc = acc + lax.gather(
                        acc,
                        ((iota + sh) & (LANES - 1))[:, None],
                        lax.GatherDimensionNumbers(
                            offset_dims=(), collapsed_slice_dims=(0,),
                            start_index_map=(0,)),
                        (1,),
                        mode=lax.GatherScatterMode.PROMISE_IN_BOUNDS)
                tot = jnp.where(iota == i, tot + acc, tot)
            sig = 1.0 / (1.0 + jnp.exp(-tot))
            out_v[pl.ds(sb * SUB + g * LANES, LANES)] = sig
            return carry2

        return lax.fori_loop(0, GROUPS, group, carry)

    lax.fori_loop(0, NSUB, sub_block, 0)
    pltpu.sync_copy(out_v, out_hbm.at[pl.ds(wid * ROWS_PER_W, ROWS_PER_W)])


@jax.jit
def _sc_call(x_r, em, w2, bvec):
    mesh = plsc.VectorSubcoreMesh(core_axis_name="c", subcore_axis_name="s",
                                  num_cores=NC, num_subcores=NS)
    fn = functools.partial(
        pl.kernel,
        out_type=jax.ShapeDtypeStruct((B,), jnp.float32),
        mesh=mesh,
        scratch_types=[
            pltpu.VMEM((NW * IDX_ROWS_PER_W // NW, CHUNK), jnp.int32),
            pltpu.VMEM((FLAT_PER_SUB, D), jnp.float32),
            pltpu.VMEM((L, D), jnp.float32),
            pltpu.VMEM((LANES,), jnp.float32),
            pltpu.VMEM((ROWS_PER_W,), jnp.float32),
            pltpu.SemaphoreType.DMA,
        ],
        compiler_params=pltpu.CompilerParams(use_tc_tiling_on_sc=False),
    )(_sc_body)
    return fn(x_r, em, w2, bvec)


def kernel(x, em, W, b):
    x_r = x.astype(jnp.int32).reshape(-1, CHUNK)
    w2 = W.reshape(L, D)
    bvec = jnp.broadcast_to(b.astype(jnp.float32), (LANES,))
    out = _sc_call(x_r, em, w2, bvec)
    return out.reshape(B, 1)
